# Initial kernel scaffold; baseline (speedup 1.0000x reference)
import jax
import jax.numpy as jnp
from jax.experimental import pallas as pl


def kernel(x, attr_idx, W1, b1, gamma, beta, W2, b2):
    raise NotImplementedError("write your pallas kernel here")


# speedup vs baseline: 1.2258x; 1.2258x over previous
import functools

import jax
import jax.numpy as jnp
from jax import lax
from jax.experimental import pallas as pl
from jax.experimental.pallas import tpu as pltpu
from jax.experimental.pallas import tpu_sc as plsc

EPS = 1e-5
BLK = 256


def _sc_row_gather(table, idx, chunk):
    m, = idx.shape
    n, d = table.shape
    info = plsc.get_sparse_core_info()
    nc, ns = info.num_cores, info.num_subcores
    nw = nc * ns
    assert m % (nw * chunk) == 0
    per_w = m // nw
    chunks = per_w // chunk

    mesh = plsc.VectorSubcoreMesh(core_axis_name="c", subcore_axis_name="s")

    @functools.partial(
        pl.kernel,
        mesh=mesh,
        out_type=jax.ShapeDtypeStruct((m, d), table.dtype),
        scratch_types=[
            pltpu.VMEM((chunk,), jnp.int32),
            pltpu.VMEM((chunk, d), table.dtype),
            pltpu.SemaphoreType.DMA,
        ],
    )
    def k(table_hbm, idx_hbm, out_hbm, idx_v, rows_v, sem):
        wid = lax.axis_index("s") * nc + lax.axis_index("c")
        base = wid * per_w
        for i in range(chunks):
            off = base + i * chunk
            pltpu.sync_copy(idx_hbm.at[pl.ds(off, chunk)], idx_v)
            pltpu.async_copy(table_hbm.at[idx_v], rows_v, sem).wait()
            pltpu.sync_copy(rows_v, out_hbm.at[pl.ds(off, chunk)])

    return k(table, idx)


def _p1_body(m_ref, xs_ref, w1_ref, b1_ref, hs_ref, sum_ref, ssq_ref):
    b = pl.program_id(0)
    h = jnp.dot(xs_ref[...], w1_ref[0], preferred_element_type=jnp.float32)
    h = h + b1_ref[0, 0][None, :]
    hs_ref[...] = h
    vc = m_ref[2, b]
    mask = (lax.broadcasted_iota(jnp.int32, h.shape, 0) < vc).astype(h.dtype)
    hm = h * mask
    ps = jnp.sum(hm, axis=0, keepdims=True)[None]
    pq = jnp.sum(hm * h, axis=0, keepdims=True)[None]

    @pl.when(m_ref[1, b] == 1)
    def _():
        sum_ref[...] = ps
        ssq_ref[...] = pq

    @pl.when(m_ref[1, b] == 0)
    def _():
        sum_ref[...] += ps
        ssq_ref[...] += pq


def _p2_body(m_ref, hs_ref, sum_ref, ssq_ref, g_ref, be_ref, w2_ref, b2_ref,
             ys_ref):
    b = pl.program_id(0)
    cnt = jnp.maximum(m_ref[1, b].astype(jnp.float32), 1.0)
    mean = sum_ref[0, 0] / cnt
    var = ssq_ref[0, 0] / cnt - mean * mean
    rstd = lax.rsqrt(var + EPS)
    scale = rstd * g_ref[0, 0]
    shift = be_ref[0, 0] - mean * scale
    h = hs_ref[...]
    a = jnp.maximum(h * scale[None, :] + shift[None, :], 0.0)
    y = jnp.dot(a, w2_ref[0], preferred_element_type=jnp.float32)
    ys_ref[...] = y + b2_ref[0, 0][None, :]


def kernel(x, attr_idx, W1, b1, gamma, beta, W2, b2):
    bsz, d = x.shape
    e_num, _, h_dim = W1.shape
    o_dim = W2.shape[2]
    nblk = bsz // BLK + e_num
    pad_b = nblk * BLK

    attr = attr_idx.astype(jnp.int32)

    order = jnp.argsort(attr).astype(jnp.int32)
    cnt = jnp.bincount(attr, length=e_num).astype(jnp.int32)
    blocks_e = (cnt + BLK - 1) // BLK
    cumblocks = jnp.cumsum(blocks_e)
    start_block = cumblocks - blocks_e
    blk_ids = jnp.arange(nblk, dtype=jnp.int32)
    eob_raw = jnp.searchsorted(cumblocks, blk_ids, side="right")
    last_e = jnp.max(jnp.where(cnt > 0, jnp.arange(e_num, dtype=jnp.int32), -1))
    eob = jnp.minimum(eob_raw.astype(jnp.int32), last_e)
    r0 = (blk_ids - start_block[eob]) * BLK
    vc = jnp.clip(cnt[eob] - r0, 0, BLK)
    is_first = (blk_ids == start_block[eob]).astype(jnp.int32)

    scnt = jnp.cumsum(cnt) - cnt
    pad_start = start_block * BLK
    p = jnp.arange(pad_b, dtype=jnp.int32)
    pe = eob[p // BLK]
    r = p - pad_start[pe]
    valid = r < cnt[pe]
    gidx = jnp.where(valid, order[jnp.clip(scnt[pe] + r, 0, bsz - 1)], 0)
    gidx = gidx.astype(jnp.int32)

    j = jnp.arange(bsz, dtype=jnp.int32)
    ej = attr[order]
    pos_sorted = pad_start[ej] - scnt[ej] + j
    pos = jnp.zeros(bsz, jnp.int32).at[order].set(pos_sorted)

    meta1 = jnp.stack([eob, is_first, vc])
    meta2 = jnp.stack([eob, cnt[eob]])

    xs = _sc_row_gather(x, gidx, chunk=64)

    grid1 = pltpu.PrefetchScalarGridSpec(
        num_scalar_prefetch=1,
        grid=(nblk,),
        in_specs=[
            pl.BlockSpec((BLK, d), lambda b, m: (b, 0)),
            pl.BlockSpec((1, d, h_dim), lambda b, m: (m[0, b], 0, 0)),
            pl.BlockSpec((1, 1, h_dim), lambda b, m: (m[0, b], 0, 0)),
        ],
        out_specs=[
            pl.BlockSpec((BLK, h_dim), lambda b, m: (b, 0)),
            pl.BlockSpec((1, 1, h_dim), lambda b, m: (m[0, b], 0, 0)),
            pl.BlockSpec((1, 1, h_dim), lambda b, m: (m[0, b], 0, 0)),
        ],
    )
    hs, sums, ssq = pl.pallas_call(
        _p1_body,
        grid_spec=grid1,
        out_shape=[
            jax.ShapeDtypeStruct((pad_b, h_dim), jnp.float32),
            jax.ShapeDtypeStruct((e_num, 1, h_dim), jnp.float32),
            jax.ShapeDtypeStruct((e_num, 1, h_dim), jnp.float32),
        ],
    )(meta1, xs, W1, b1.reshape(e_num, 1, h_dim))

    grid2 = pltpu.PrefetchScalarGridSpec(
        num_scalar_prefetch=1,
        grid=(nblk,),
        in_specs=[
            pl.BlockSpec((BLK, h_dim), lambda b, m: (b, 0)),
            pl.BlockSpec((1, 1, h_dim), lambda b, m: (m[0, b], 0, 0)),
            pl.BlockSpec((1, 1, h_dim), lambda b, m: (m[0, b], 0, 0)),
            pl.BlockSpec((1, 1, h_dim), lambda b, m: (m[0, b], 0, 0)),
            pl.BlockSpec((1, 1, h_dim), lambda b, m: (m[0, b], 0, 0)),
            pl.BlockSpec((1, h_dim, o_dim), lambda b, m: (m[0, b], 0, 0)),
            pl.BlockSpec((1, 1, o_dim), lambda b, m: (m[0, b], 0, 0)),
        ],
        out_specs=[
            pl.BlockSpec((BLK, o_dim), lambda b, m: (b, 0)),
        ],
    )
    ys, = pl.pallas_call(
        _p2_body,
        grid_spec=grid2,
        out_shape=[jax.ShapeDtypeStruct((pad_b, o_dim), jnp.float32)],
    )(meta2, hs, sums, ssq, gamma.reshape(e_num, 1, h_dim),
      beta.reshape(e_num, 1, h_dim), W2, b2.reshape(e_num, 1, o_dim))

    return _sc_row_gather(ys, pos, chunk=64)
